# Initial kernel scaffold; baseline (speedup 1.0000x reference)
import jax
import jax.numpy as jnp
from jax.experimental import pallas as pl


def kernel(x, edge_index, batch, W1, b1, W2, b2, eps):
    raise NotImplementedError("write your pallas kernel here")


# speedup vs baseline: 5.6923x; 5.6923x over previous
import functools

import jax
import jax.numpy as jnp
from jax import lax
from jax.experimental import pallas as pl
from jax.experimental.pallas import tpu as pltpu
from jax.experimental.pallas import tpu_sc as plsc

N = 10000
E = 320000
D = 128
G = 64

NC = 2
NS = 16
LANES = 16

NW = NC * NS
EPT = E // NW
CH = 128
NFULL = EPT // CH
TAIL = EPT - NFULL * CH

ROWS_A = 624
ROWS_B = N - 15 * ROWS_A


def _zero_fill(ref, nrows):
    def body(i, c):
        for j in range(D // LANES):
            ref[i, pl.ds(j * LANES, LANES)] = jnp.zeros((LANES,), jnp.float32)
        return c
    lax.fori_loop(0, nrows, body, 0)


def _one_fill(ref, nrows):
    def body(i, c):
        for j in range(D // LANES):
            ref[i, pl.ds(j * LANES, LANES)] = jnp.ones((LANES,), jnp.float32)
        return c
    lax.fori_loop(0, nrows, body, 0)


@functools.partial(
    pl.kernel,
    out_type=jax.ShapeDtypeStruct((2 * N, D), jnp.float32),
    mesh=plsc.VectorSubcoreMesh(core_axis_name="c", subcore_axis_name="s"),
    scratch_types=[
        pltpu.VMEM((CH, D), jnp.float32),
        pltpu.VMEM((CH,), jnp.int32),
        pltpu.VMEM((CH,), jnp.int32),
        pltpu.VMEM((TAIL, D), jnp.float32),
        pltpu.VMEM((TAIL,), jnp.int32),
        pltpu.VMEM((TAIL,), jnp.int32),
        pltpu.VMEM((CH, D), jnp.float32),
        pltpu.VMEM_SHARED((N, D), jnp.float32),
        pltpu.SemaphoreType.DMA,
    ],
)
def _segsum(x_hbm, src_hbm, dst_hbm, out_hbm,
            rows, sidx, didx, trows, tsidx, tdidx, zbuf, acc, sem):
    cid = lax.axis_index("c")
    sid = lax.axis_index("s")
    wid = sid * NC + cid

    _zero_fill(zbuf, CH)
    rbase = sid * ROWS_A

    def zcopy(k, c):
        pltpu.sync_copy(zbuf, acc.at[pl.ds(rbase + k * CH, CH)])
        return c
    lax.fori_loop(0, 4, zcopy, 0)

    @pl.when(sid == NS - 1)
    def _():
        zcopy(4, 0)

    @pl.when(sid < NS - 1)
    def _():
        pltpu.sync_copy(zbuf.at[pl.ds(0, ROWS_A - 4 * CH)],
                        acc.at[pl.ds(rbase + 4 * CH, ROWS_A - 4 * CH)])

    plsc.subcore_barrier()

    ebase = wid * EPT

    def chunk(j, c):
        base = ebase + j * CH
        pltpu.sync_copy(src_hbm.at[pl.ds(base, CH)], sidx)
        pltpu.sync_copy(dst_hbm.at[pl.ds(base, CH)], didx)
        pltpu.async_copy(x_hbm.at[sidx], rows, sem).wait()
        pltpu.sync_copy(rows, acc.at[didx], add=True)
        return c
    lax.fori_loop(0, NFULL, chunk, 0)

    tbase = ebase + NFULL * CH
    pltpu.sync_copy(src_hbm.at[pl.ds(tbase, TAIL)], tsidx)
    pltpu.sync_copy(dst_hbm.at[pl.ds(tbase, TAIL)], tdidx)
    pltpu.async_copy(x_hbm.at[tsidx], trows, sem).wait()
    pltpu.sync_copy(trows, acc.at[tdidx], add=True)

    plsc.subcore_barrier()

    @pl.when(sid < NS - 1)
    def _():
        r0 = sid * ROWS_A
        pltpu.sync_copy(acc.at[pl.ds(r0, ROWS_A)],
                        out_hbm.at[pl.ds(cid * N + r0, ROWS_A)])

    @pl.when(sid == NS - 1)
    def _():
        r0 = (NS - 1) * ROWS_A
        pltpu.sync_copy(acc.at[pl.ds(r0, ROWS_B)],
                        out_hbm.at[pl.ds(cid * N + r0, ROWS_B)])


BR = 1000


def _mlp_body(scale_ref, cur_ref, agg_ref, w1_ref, b1_ref, w2_ref, b2_ref,
              out_ref, *, out_relu):
    h = scale_ref[0, 0] * cur_ref[...] + agg_ref[0] + agg_ref[1]
    t = jnp.dot(h, w1_ref[...], preferred_element_type=jnp.float32) + b1_ref[...]
    t = jnp.maximum(t, 0.0)
    o = jnp.dot(t, w2_ref[...], preferred_element_type=jnp.float32) + b2_ref[...]
    if out_relu:
        o = jnp.maximum(o, 0.0)
    out_ref[...] = o


def _mlp3_body(scale_ref, cur_ref, agg_ref, w1_ref, b1_ref, w2_ref, b2_ref,
               c1_ref, c2_ref, z_ref):
    h = scale_ref[0, 0] * cur_ref[...] + agg_ref[0] + agg_ref[1]
    t = jnp.dot(h, w1_ref[...], preferred_element_type=jnp.float32) + b1_ref[...]
    t = jnp.maximum(t, 0.0)
    o = jnp.dot(t, w2_ref[...], preferred_element_type=jnp.float32) + b2_ref[...]
    z_ref[...] = (c1_ref[...] + c2_ref[...] + o) * (1.0 / 3.0)


_scale_spec = pl.BlockSpec((1, 1), lambda i: (0, 0), memory_space=pltpu.SMEM)
_row_spec = pl.BlockSpec((BR, D), lambda i: (i, 0))
_agg_spec = pl.BlockSpec((2, BR, D), lambda i: (0, i, 0))
_w_spec = pl.BlockSpec((D, D), lambda i: (0, 0))
_b_spec = pl.BlockSpec((1, D), lambda i: (0, 0))


def _mlp(cur, agg2, w1, b1, w2, b2, scale, out_relu):
    body = functools.partial(_mlp_body, out_relu=out_relu)
    return pl.pallas_call(
        body,
        grid=(N // BR,),
        in_specs=[_scale_spec, _row_spec, _agg_spec,
                  _w_spec, _b_spec, _w_spec, _b_spec],
        out_specs=_row_spec,
        out_shape=jax.ShapeDtypeStruct((N, D), jnp.float32),
        compiler_params=pltpu.CompilerParams(
            dimension_semantics=("arbitrary",)),
    )(scale, cur, agg2, w1, b1.reshape(1, D), w2, b2.reshape(1, D))


def _mlp3(cur, agg2, w1, b1, w2, b2, scale, c1, c2):
    return pl.pallas_call(
        _mlp3_body,
        grid=(N // BR,),
        in_specs=[_scale_spec, _row_spec, _agg_spec,
                  _w_spec, _b_spec, _w_spec, _b_spec,
                  _row_spec, _row_spec],
        out_specs=_row_spec,
        out_shape=jax.ShapeDtypeStruct((N, D), jnp.float32),
        compiler_params=pltpu.CompilerParams(
            dimension_semantics=("arbitrary",)),
    )(scale, cur, agg2, w1, b1.reshape(1, D), w2, b2.reshape(1, D), c1, c2)


PCH = 128
PTAILR = ROWS_A - 4 * PCH


@functools.partial(
    pl.kernel,
    out_type=jax.ShapeDtypeStruct((G, D), jnp.float32),
    mesh=plsc.VectorSubcoreMesh(core_axis_name="c", subcore_axis_name="s"),
    scratch_types=[
        pltpu.VMEM((PCH, D), jnp.float32),
        pltpu.VMEM((PCH, D), jnp.float32),
        pltpu.VMEM((PCH,), jnp.int32),
        pltpu.VMEM((PTAILR,), jnp.int32),
        pltpu.VMEM((G, D), jnp.float32),
        pltpu.VMEM((G, D), jnp.float32),
        pltpu.VMEM_SHARED((G, D), jnp.float32),
        pltpu.VMEM_SHARED((G, D), jnp.float32),
    ],
)
def _pool(z_hbm, batch_hbm, g_hbm,
          rows, ones, bidx, bidxt, gsum, gcnt, accs, accc):
    cid = lax.axis_index("c")
    sid = lax.axis_index("s")

    @pl.when(cid == 0)
    def _():
        _one_fill(ones, PCH)

    @pl.when((cid == 0) & (sid == 0))
    def _():
        _zero_fill(gsum, G)
        pltpu.sync_copy(gsum, accs)
        pltpu.sync_copy(gsum, accc)

    plsc.subcore_barrier()

    @pl.when(cid == 0)
    def _():
        rbase = sid * ROWS_A

        def chunk(k, c):
            base = rbase + k * PCH
            pltpu.sync_copy(batch_hbm.at[pl.ds(base, PCH)], bidx)
            pltpu.sync_copy(z_hbm.at[pl.ds(base, PCH)], rows)
            pltpu.sync_copy(rows, accs.at[bidx], add=True)
            pltpu.sync_copy(ones, accc.at[bidx], add=True)
            return c
        lax.fori_loop(0, 4, chunk, 0)

        @pl.when(sid == NS - 1)
        def _():
            chunk(4, 0)

        @pl.when(sid < NS - 1)
        def _():
            base = rbase + 4 * PCH
            pltpu.sync_copy(batch_hbm.at[pl.ds(base, PTAILR)], bidxt)
            pltpu.sync_copy(z_hbm.at[pl.ds(base, PTAILR)],
                            rows.at[pl.ds(0, PTAILR)])
            pltpu.sync_copy(rows.at[pl.ds(0, PTAILR)], accs.at[bidxt], add=True)
            pltpu.sync_copy(ones.at[pl.ds(0, PTAILR)], accc.at[bidxt], add=True)

    plsc.subcore_barrier()

    @pl.when((cid == 0) & (sid == 0))
    def _():
        pltpu.sync_copy(accs, gsum)
        pltpu.sync_copy(accc, gcnt)

        def fin(i, c):
            for j in range(D // LANES):
                s = gsum[i, pl.ds(j * LANES, LANES)]
                n = gcnt[i, pl.ds(j * LANES, LANES)]
                gsum[i, pl.ds(j * LANES, LANES)] = s / jnp.maximum(n, 1.0)
            return c
        lax.fori_loop(0, G, fin, 0)
        pltpu.sync_copy(gsum, g_hbm)


def kernel(x, edge_index, batch, W1, b1, W2, b2, eps):
    src = edge_index[0]
    dst = edge_index[1]

    agg1 = _segsum(x, src, dst).reshape(2, N, D)
    cur1 = _mlp(x, agg1, W1[0], b1[0], W2[0], b2[0],
                (1.0 + eps[0]).reshape(1, 1), out_relu=True)
    agg2 = _segsum(cur1, src, dst).reshape(2, N, D)
    cur2 = _mlp(cur1, agg2, W1[1], b1[1], W2[1], b2[1],
                (1.0 + eps[1]).reshape(1, 1), out_relu=True)
    agg3 = _segsum(cur2, src, dst).reshape(2, N, D)
    z = _mlp3(cur2, agg3, W1[2], b1[2], W2[2], b2[2],
              (1.0 + eps[2]).reshape(1, 1), cur1, cur2)
    g = _pool(z, batch)
    return (z, g)


# speedup vs baseline: 11.7030x; 2.0559x over previous
import functools

import jax
import jax.numpy as jnp
from jax import lax
from jax.experimental import pallas as pl
from jax.experimental.pallas import tpu as pltpu
from jax.experimental.pallas import tpu_sc as plsc

N = 10000
E = 320000
D = 128
G = 64

NC = 2
NS = 16
LANES = 16

NW = NC * NS
CH = 128
NCHUNK = E // CH
CPT = NCHUNK // NW
XCH = NCHUNK - CPT * NW

ROWS_A = 624
ROWS_B = N - 15 * ROWS_A
ZR = 64


def _zero_fill(ref, nrows):
    def body(i, c):
        for j in range(D // LANES):
            ref[i, pl.ds(j * LANES, LANES)] = jnp.zeros((LANES,), jnp.float32)
        return c
    lax.fori_loop(0, nrows, body, 0)


def _one_fill(ref, nrows):
    def body(i, c):
        for j in range(D // LANES):
            ref[i, pl.ds(j * LANES, LANES)] = jnp.ones((LANES,), jnp.float32)
        return c
    lax.fori_loop(0, nrows, body, 0)


@functools.partial(
    pl.kernel,
    out_type=jax.ShapeDtypeStruct((2 * N, D), jnp.float32),
    mesh=plsc.VectorSubcoreMesh(core_axis_name="c", subcore_axis_name="s"),
    scratch_types=[
        pltpu.VMEM((CH, D), jnp.float32),
        pltpu.VMEM((CH, D), jnp.float32),
        pltpu.VMEM((2, 1, CH), jnp.int32),
        pltpu.VMEM((2, 1, CH), jnp.int32),
        pltpu.VMEM((2, 1, CH), jnp.int32),
        pltpu.VMEM((2, 1, CH), jnp.int32),
        pltpu.VMEM((ZR, D), jnp.float32),
        pltpu.VMEM_SHARED((N, D), jnp.float32),
        pltpu.SemaphoreType.DMA,
        pltpu.SemaphoreType.DMA,
        pltpu.SemaphoreType.DMA,
        pltpu.SemaphoreType.DMA,
        pltpu.SemaphoreType.DMA,
        pltpu.SemaphoreType.DMA,
        pltpu.SemaphoreType.DMA,
        pltpu.SemaphoreType.DMA,
    ],
)
def _segsum(x_hbm, ei_hbm, out_hbm,
            rows0, rows1, ib0, ib1, ib2, ib3, zbuf, acc,
            isem0, isem1, isem2, isem3, gsem0, gsem1, ssem0, ssem1):
    cid = lax.axis_index("c")
    sid = lax.axis_index("s")
    wid = sid * NC + cid

    rows = (rows0, rows1)
    ibs = (ib0, ib1, ib2, ib3)
    isems = (isem0, isem1, isem2, isem3)
    gsems = (gsem0, gsem1)
    ssems = (ssem0, ssem1)

    def idx_desc(c, p4):
        return pltpu.make_async_copy(
            ei_hbm.at[:, pl.ds(c, 1), :], ibs[p4], isems[p4])

    def gather_desc(p4, p2):
        return pltpu.make_async_copy(
            x_hbm.at[ibs[p4].at[0, 0]], rows[p2], gsems[p2])

    def scatter_desc(p4, p2):
        return pltpu.make_async_copy(
            rows[p2], acc.at[ibs[p4].at[1, 0]], ssems[p2])

    _zero_fill(zbuf, ZR)
    rbase = sid * ROWS_A

    def zcopy(k, c):
        pltpu.sync_copy(zbuf, acc.at[pl.ds(rbase + k * ZR, ZR)])
        return c
    lax.fori_loop(0, 9, zcopy, 0)

    @pl.when(sid == NS - 1)
    def _():
        zcopy(9, 0)

    @pl.when(sid < NS - 1)
    def _():
        pltpu.sync_copy(zbuf.at[pl.ds(0, ROWS_A - 9 * ZR)],
                        acc.at[pl.ds(rbase + 9 * ZR, ROWS_A - 9 * ZR)])

    plsc.subcore_barrier()

    cb = wid * CPT

    def body(c, j, fire_next=True, drain_prev2=True):
        p2, p4 = j % 2, j % 4
        if drain_prev2:
            scatter_desc((j - 2) % 4, p2).wait()
        if fire_next:
            idx_desc(c + 1, (j + 1) % 4).start()
        idx_desc(c, p4).wait()
        gather_desc(p4, p2).start()
        gather_desc((j - 1) % 4, 1 - p2).wait()
        scatter_desc((j - 1) % 4, 1 - p2).start(add=True)

    idx_desc(cb, 0).start()
    idx_desc(cb + 1, 1).start()
    idx_desc(cb, 0).wait()
    gather_desc(0, 0).start()
    idx_desc(cb + 2, 2).start()
    idx_desc(cb + 1, 1).wait()
    gather_desc(1, 1).start()
    gather_desc(0, 0).wait()
    scatter_desc(0, 0).start(add=True)

    def quad(i, carry):
        c0 = cb + 2 + 4 * i
        for t in range(4):
            body(c0 + t, 2 + t)
        return carry
    lax.fori_loop(0, (CPT - 2) // 4, quad, 0)

    jl = CPT - 1
    gather_desc(jl % 4, jl % 2).wait()
    scatter_desc(jl % 4, jl % 2).start(add=True)
    scatter_desc((jl - 1) % 4, (jl - 1) % 2).wait()
    scatter_desc(jl % 4, jl % 2).wait()

    @pl.when(wid < XCH)
    def _():
        cx = NCHUNK - XCH + wid
        idx_desc(cx, 0).start()
        idx_desc(cx, 0).wait()
        gather_desc(0, 0).start()
        gather_desc(0, 0).wait()
        scatter_desc(0, 0).start(add=True)
        scatter_desc(0, 0).wait()

    plsc.subcore_barrier()

    @pl.when(sid < NS - 1)
    def _():
        r0 = sid * ROWS_A
        pltpu.sync_copy(acc.at[pl.ds(r0, ROWS_A)],
                        out_hbm.at[pl.ds(cid * N + r0, ROWS_A)])

    @pl.when(sid == NS - 1)
    def _():
        r0 = (NS - 1) * ROWS_A
        pltpu.sync_copy(acc.at[pl.ds(r0, ROWS_B)],
                        out_hbm.at[pl.ds(cid * N + r0, ROWS_B)])


BR = 1000


def _mlp_body(scale_ref, cur_ref, agg_ref, w1_ref, b1_ref, w2_ref, b2_ref,
              out_ref, *, out_relu):
    h = scale_ref[0, 0] * cur_ref[...] + agg_ref[0] + agg_ref[1]
    t = jnp.dot(h, w1_ref[...], preferred_element_type=jnp.float32) + b1_ref[...]
    t = jnp.maximum(t, 0.0)
    o = jnp.dot(t, w2_ref[...], preferred_element_type=jnp.float32) + b2_ref[...]
    if out_relu:
        o = jnp.maximum(o, 0.0)
    out_ref[...] = o


def _mlp3_body(scale_ref, cur_ref, agg_ref, w1_ref, b1_ref, w2_ref, b2_ref,
               c1_ref, c2_ref, z_ref):
    h = scale_ref[0, 0] * cur_ref[...] + agg_ref[0] + agg_ref[1]
    t = jnp.dot(h, w1_ref[...], preferred_element_type=jnp.float32) + b1_ref[...]
    t = jnp.maximum(t, 0.0)
    o = jnp.dot(t, w2_ref[...], preferred_element_type=jnp.float32) + b2_ref[...]
    z_ref[...] = (c1_ref[...] + c2_ref[...] + o) * (1.0 / 3.0)


_scale_spec = pl.BlockSpec((1, 1), lambda i: (0, 0), memory_space=pltpu.SMEM)
_row_spec = pl.BlockSpec((BR, D), lambda i: (i, 0))
_agg_spec = pl.BlockSpec((2, BR, D), lambda i: (0, i, 0))
_w_spec = pl.BlockSpec((D, D), lambda i: (0, 0))
_b_spec = pl.BlockSpec((1, D), lambda i: (0, 0))


def _mlp(cur, agg2, w1, b1, w2, b2, scale, out_relu):
    body = functools.partial(_mlp_body, out_relu=out_relu)
    return pl.pallas_call(
        body,
        grid=(N // BR,),
        in_specs=[_scale_spec, _row_spec, _agg_spec,
                  _w_spec, _b_spec, _w_spec, _b_spec],
        out_specs=_row_spec,
        out_shape=jax.ShapeDtypeStruct((N, D), jnp.float32),
        compiler_params=pltpu.CompilerParams(
            dimension_semantics=("arbitrary",)),
    )(scale, cur, agg2, w1, b1.reshape(1, D), w2, b2.reshape(1, D))


def _mlp3(cur, agg2, w1, b1, w2, b2, scale, c1, c2):
    return pl.pallas_call(
        _mlp3_body,
        grid=(N // BR,),
        in_specs=[_scale_spec, _row_spec, _agg_spec,
                  _w_spec, _b_spec, _w_spec, _b_spec,
                  _row_spec, _row_spec],
        out_specs=_row_spec,
        out_shape=jax.ShapeDtypeStruct((N, D), jnp.float32),
        compiler_params=pltpu.CompilerParams(
            dimension_semantics=("arbitrary",)),
    )(scale, cur, agg2, w1, b1.reshape(1, D), w2, b2.reshape(1, D), c1, c2)


PCH = 128
PTAILR = ROWS_A - 4 * PCH


@functools.partial(
    pl.kernel,
    out_type=jax.ShapeDtypeStruct((G, D), jnp.float32),
    mesh=plsc.VectorSubcoreMesh(core_axis_name="c", subcore_axis_name="s"),
    scratch_types=[
        pltpu.VMEM((PCH, D), jnp.float32),
        pltpu.VMEM((PCH, D), jnp.float32),
        pltpu.VMEM((PCH,), jnp.int32),
        pltpu.VMEM((PTAILR,), jnp.int32),
        pltpu.VMEM((G, D), jnp.float32),
        pltpu.VMEM((G, D), jnp.float32),
        pltpu.VMEM_SHARED((G, D), jnp.float32),
        pltpu.VMEM_SHARED((G, D), jnp.float32),
    ],
)
def _pool(z_hbm, batch_hbm, g_hbm,
          rows, ones, bidx, bidxt, gsum, gcnt, accs, accc):
    cid = lax.axis_index("c")
    sid = lax.axis_index("s")

    @pl.when(cid == 0)
    def _():
        _one_fill(ones, PCH)

    @pl.when((cid == 0) & (sid == 0))
    def _():
        _zero_fill(gsum, G)
        pltpu.sync_copy(gsum, accs)
        pltpu.sync_copy(gsum, accc)

    plsc.subcore_barrier()

    @pl.when(cid == 0)
    def _():
        rbase = sid * ROWS_A

        def chunk(k, c):
            base = rbase + k * PCH
            pltpu.sync_copy(batch_hbm.at[pl.ds(base, PCH)], bidx)
            pltpu.sync_copy(z_hbm.at[pl.ds(base, PCH)], rows)
            pltpu.sync_copy(rows, accs.at[bidx], add=True)
            pltpu.sync_copy(ones, accc.at[bidx], add=True)
            return c
        lax.fori_loop(0, 4, chunk, 0)

        @pl.when(sid == NS - 1)
        def _():
            chunk(4, 0)

        @pl.when(sid < NS - 1)
        def _():
            base = rbase + 4 * PCH
            pltpu.sync_copy(batch_hbm.at[pl.ds(base, PTAILR)], bidxt)
            pltpu.sync_copy(z_hbm.at[pl.ds(base, PTAILR)],
                            rows.at[pl.ds(0, PTAILR)])
            pltpu.sync_copy(rows.at[pl.ds(0, PTAILR)], accs.at[bidxt], add=True)
            pltpu.sync_copy(ones.at[pl.ds(0, PTAILR)], accc.at[bidxt], add=True)

    plsc.subcore_barrier()

    @pl.when((cid == 0) & (sid == 0))
    def _():
        pltpu.sync_copy(accs, gsum)
        pltpu.sync_copy(accc, gcnt)

        def fin(i, c):
            for j in range(D // LANES):
                s = gsum[i, pl.ds(j * LANES, LANES)]
                n = gcnt[i, pl.ds(j * LANES, LANES)]
                gsum[i, pl.ds(j * LANES, LANES)] = s / jnp.maximum(n, 1.0)
            return c
        lax.fori_loop(0, G, fin, 0)
        pltpu.sync_copy(gsum, g_hbm)


def kernel(x, edge_index, batch, W1, b1, W2, b2, eps):
    ei3 = edge_index.reshape(2, NCHUNK, CH)

    agg1 = _segsum(x, ei3).reshape(2, N, D)
    cur1 = _mlp(x, agg1, W1[0], b1[0], W2[0], b2[0],
                (1.0 + eps[0]).reshape(1, 1), out_relu=True)
    agg2 = _segsum(cur1, ei3).reshape(2, N, D)
    cur2 = _mlp(cur1, agg2, W1[1], b1[1], W2[1], b2[1],
                (1.0 + eps[1]).reshape(1, 1), out_relu=True)
    agg3 = _segsum(cur2, ei3).reshape(2, N, D)
    z = _mlp3(cur2, agg3, W1[2], b1[2], W2[2], b2[2],
              (1.0 + eps[2]).reshape(1, 1), cur1, cur2)
    g = _pool(z, batch)
    return (z, g)


# speedup vs baseline: 12.3542x; 1.0556x over previous
import functools

import jax
import jax.numpy as jnp
from jax import lax
from jax.experimental import pallas as pl
from jax.experimental.pallas import tpu as pltpu
from jax.experimental.pallas import tpu_sc as plsc

N = 10000
E = 320000
D = 128
G = 64

NC = 2
NS = 16
LANES = 16

NW = NC * NS
CH = 128
NCHUNK = E // CH
CPT = NCHUNK // NW
XCH = NCHUNK - CPT * NW

ROWS_A = 624
ROWS_B = N - 15 * ROWS_A
ZR = 64


def _zero_fill(ref, nrows):
    def body(i, c):
        for j in range(D // LANES):
            ref[i, pl.ds(j * LANES, LANES)] = jnp.zeros((LANES,), jnp.float32)
        return c
    lax.fori_loop(0, nrows, body, 0)


def _one_fill(ref, nrows):
    def body(i, c):
        for j in range(D // LANES):
            ref[i, pl.ds(j * LANES, LANES)] = jnp.ones((LANES,), jnp.float32)
        return c
    lax.fori_loop(0, nrows, body, 0)


@functools.partial(
    pl.kernel,
    out_type=jax.ShapeDtypeStruct((2 * N, D), jnp.float32),
    mesh=plsc.VectorSubcoreMesh(core_axis_name="c", subcore_axis_name="s"),
    scratch_types=[
        pltpu.VMEM((CH, D), jnp.float32),
        pltpu.VMEM((CH, D), jnp.float32),
        pltpu.VMEM((CH, D), jnp.float32),
        pltpu.VMEM((2, 1, CH), jnp.int32),
        pltpu.VMEM((2, 1, CH), jnp.int32),
        pltpu.VMEM((2, 1, CH), jnp.int32),
        pltpu.VMEM((2, 1, CH), jnp.int32),
        pltpu.VMEM_SHARED((N, D), jnp.float32),
        pltpu.SemaphoreType.DMA,
        pltpu.SemaphoreType.DMA,
        pltpu.SemaphoreType.DMA,
        pltpu.SemaphoreType.DMA,
        pltpu.SemaphoreType.DMA,
        pltpu.SemaphoreType.DMA,
        pltpu.SemaphoreType.DMA,
        pltpu.SemaphoreType.DMA,
        pltpu.SemaphoreType.DMA,
        pltpu.SemaphoreType.DMA,
    ],
)
def _segsum(x_hbm, ei_hbm, out_hbm,
            rows0, rows1, rows2, ib0, ib1, ib2, ib3, acc,
            isem0, isem1, isem2, isem3, gsem0, gsem1, gsem2,
            ssem0, ssem1, ssem2):
    cid = lax.axis_index("c")
    sid = lax.axis_index("s")
    wid = sid * NC + cid

    rows = (rows0, rows1, rows2)
    ibs = (ib0, ib1, ib2, ib3)
    isems = (isem0, isem1, isem2, isem3)
    gsems = (gsem0, gsem1, gsem2)
    ssems = (ssem0, ssem1, ssem2)

    def idx_desc(c, p4):
        return pltpu.make_async_copy(
            ei_hbm.at[:, pl.ds(c, 1), :], ibs[p4], isems[p4])

    def gather_desc(p4, p3):
        return pltpu.make_async_copy(
            x_hbm.at[ibs[p4].at[0, 0]], rows[p3], gsems[p3])

    def scatter_desc(p4, p3):
        return pltpu.make_async_copy(
            rows[p3], acc.at[ibs[p4].at[1, 0]], ssems[p3])

    _zero_fill(rows0, CH)
    rbase = sid * ROWS_A

    def zcopy(k, c):
        pltpu.sync_copy(rows0, acc.at[pl.ds(rbase + k * CH, CH)])
        return c
    lax.fori_loop(0, 4, zcopy, 0)

    @pl.when(sid == NS - 1)
    def _():
        zcopy(4, 0)

    @pl.when(sid < NS - 1)
    def _():
        pltpu.sync_copy(rows0.at[pl.ds(0, ROWS_A - 4 * CH)],
                        acc.at[pl.ds(rbase + 4 * CH, ROWS_A - 4 * CH)])

    plsc.subcore_barrier()

    cb = wid * CPT

    def body(c, j, drain=True, scat=True, fire_next=True):
        p3, p4 = j % 3, j % 4
        if drain:
            scatter_desc((j - 3) % 4, p3).wait()
        if fire_next:
            idx_desc(c + 1, (j + 1) % 4).start()
        idx_desc(c, p4).wait()
        gather_desc(p4, p3).start()
        if scat:
            gather_desc((j - 1) % 4, (j - 1) % 3).wait()
            scatter_desc((j - 1) % 4, (j - 1) % 3).start(add=True)

    idx_desc(cb, 0).start()
    body(cb + 0, 0, drain=False, scat=False)
    body(cb + 1, 1, drain=False)
    body(cb + 2, 2, drain=False)
    body(cb + 3, 3)
    body(cb + 4, 4)
    body(cb + 5, 5)

    def twelve(i, carry):
        c0 = cb + 6 + 12 * i
        for t in range(12):
            body(c0 + t, 6 + t)
        return carry
    lax.fori_loop(0, (CPT - 6) // 12, twelve, 0)

    jl = CPT - 1
    idx_desc(cb, (jl + 1) % 4).wait()
    gather_desc(jl % 4, jl % 3).wait()
    scatter_desc(jl % 4, jl % 3).start(add=True)
    scatter_desc((jl - 2) % 4, (jl - 2) % 3).wait()
    scatter_desc((jl - 1) % 4, (jl - 1) % 3).wait()
    scatter_desc(jl % 4, jl % 3).wait()

    @pl.when(wid < XCH)
    def _():
        cx = NCHUNK - XCH + wid
        idx_desc(cx, 0).start()
        idx_desc(cx, 0).wait()
        gather_desc(0, 0).start()
        gather_desc(0, 0).wait()
        scatter_desc(0, 0).start(add=True)
        scatter_desc(0, 0).wait()

    plsc.subcore_barrier()

    @pl.when(sid < NS - 1)
    def _():
        r0 = sid * ROWS_A
        pltpu.sync_copy(acc.at[pl.ds(r0, ROWS_A)],
                        out_hbm.at[pl.ds(cid * N + r0, ROWS_A)])

    @pl.when(sid == NS - 1)
    def _():
        r0 = (NS - 1) * ROWS_A
        pltpu.sync_copy(acc.at[pl.ds(r0, ROWS_B)],
                        out_hbm.at[pl.ds(cid * N + r0, ROWS_B)])


BR = 1000


def _mlp_body(scale_ref, cur_ref, agg_ref, w1_ref, b1_ref, w2_ref, b2_ref,
              out_ref, *, out_relu):
    h = scale_ref[0, 0] * cur_ref[...] + agg_ref[0] + agg_ref[1]
    t = jnp.dot(h, w1_ref[...], preferred_element_type=jnp.float32) + b1_ref[...]
    t = jnp.maximum(t, 0.0)
    o = jnp.dot(t, w2_ref[...], preferred_element_type=jnp.float32) + b2_ref[...]
    if out_relu:
        o = jnp.maximum(o, 0.0)
    out_ref[...] = o


def _mlp3_body(scale_ref, cur_ref, agg_ref, w1_ref, b1_ref, w2_ref, b2_ref,
               c1_ref, c2_ref, z_ref):
    h = scale_ref[0, 0] * cur_ref[...] + agg_ref[0] + agg_ref[1]
    t = jnp.dot(h, w1_ref[...], preferred_element_type=jnp.float32) + b1_ref[...]
    t = jnp.maximum(t, 0.0)
    o = jnp.dot(t, w2_ref[...], preferred_element_type=jnp.float32) + b2_ref[...]
    z_ref[...] = (c1_ref[...] + c2_ref[...] + o) * (1.0 / 3.0)


_scale_spec = pl.BlockSpec((1, 1), lambda i: (0, 0), memory_space=pltpu.SMEM)
_row_spec = pl.BlockSpec((BR, D), lambda i: (i, 0))
_agg_spec = pl.BlockSpec((2, BR, D), lambda i: (0, i, 0))
_w_spec = pl.BlockSpec((D, D), lambda i: (0, 0))
_b_spec = pl.BlockSpec((1, D), lambda i: (0, 0))


def _mlp(cur, agg2, w1, b1, w2, b2, scale, out_relu):
    body = functools.partial(_mlp_body, out_relu=out_relu)
    return pl.pallas_call(
        body,
        grid=(N // BR,),
        in_specs=[_scale_spec, _row_spec, _agg_spec,
                  _w_spec, _b_spec, _w_spec, _b_spec],
        out_specs=_row_spec,
        out_shape=jax.ShapeDtypeStruct((N, D), jnp.float32),
        compiler_params=pltpu.CompilerParams(
            dimension_semantics=("arbitrary",)),
    )(scale, cur, agg2, w1, b1.reshape(1, D), w2, b2.reshape(1, D))


def _mlp3(cur, agg2, w1, b1, w2, b2, scale, c1, c2):
    return pl.pallas_call(
        _mlp3_body,
        grid=(N // BR,),
        in_specs=[_scale_spec, _row_spec, _agg_spec,
                  _w_spec, _b_spec, _w_spec, _b_spec,
                  _row_spec, _row_spec],
        out_specs=_row_spec,
        out_shape=jax.ShapeDtypeStruct((N, D), jnp.float32),
        compiler_params=pltpu.CompilerParams(
            dimension_semantics=("arbitrary",)),
    )(scale, cur, agg2, w1, b1.reshape(1, D), w2, b2.reshape(1, D), c1, c2)


PCH = 128
PTAILR = ROWS_A - 4 * PCH


@functools.partial(
    pl.kernel,
    out_type=jax.ShapeDtypeStruct((G, D), jnp.float32),
    mesh=plsc.VectorSubcoreMesh(core_axis_name="c", subcore_axis_name="s"),
    scratch_types=[
        pltpu.VMEM((PCH, D), jnp.float32),
        pltpu.VMEM((PCH, D), jnp.float32),
        pltpu.VMEM((PCH,), jnp.int32),
        pltpu.VMEM((PTAILR,), jnp.int32),
        pltpu.VMEM((G, D), jnp.float32),
        pltpu.VMEM((G, D), jnp.float32),
        pltpu.VMEM_SHARED((G, D), jnp.float32),
        pltpu.VMEM_SHARED((G, D), jnp.float32),
    ],
)
def _pool(z_hbm, batch_hbm, g_hbm,
          rows, ones, bidx, bidxt, gsum, gcnt, accs, accc):
    cid = lax.axis_index("c")
    sid = lax.axis_index("s")

    @pl.when(cid == 0)
    def _():
        _one_fill(ones, PCH)

    @pl.when((cid == 0) & (sid == 0))
    def _():
        _zero_fill(gsum, G)
        pltpu.sync_copy(gsum, accs)
        pltpu.sync_copy(gsum, accc)

    plsc.subcore_barrier()

    @pl.when(cid == 0)
    def _():
        rbase = sid * ROWS_A

        def chunk(k, c):
            base = rbase + k * PCH
            pltpu.sync_copy(batch_hbm.at[pl.ds(base, PCH)], bidx)
            pltpu.sync_copy(z_hbm.at[pl.ds(base, PCH)], rows)
            pltpu.sync_copy(rows, accs.at[bidx], add=True)
            pltpu.sync_copy(ones, accc.at[bidx], add=True)
            return c
        lax.fori_loop(0, 4, chunk, 0)

        @pl.when(sid == NS - 1)
        def _():
            chunk(4, 0)

        @pl.when(sid < NS - 1)
        def _():
            base = rbase + 4 * PCH
            pltpu.sync_copy(batch_hbm.at[pl.ds(base, PTAILR)], bidxt)
            pltpu.sync_copy(z_hbm.at[pl.ds(base, PTAILR)],
                            rows.at[pl.ds(0, PTAILR)])
            pltpu.sync_copy(rows.at[pl.ds(0, PTAILR)], accs.at[bidxt], add=True)
            pltpu.sync_copy(ones.at[pl.ds(0, PTAILR)], accc.at[bidxt], add=True)

    plsc.subcore_barrier()

    @pl.when((cid == 0) & (sid == 0))
    def _():
        pltpu.sync_copy(accs, gsum)
        pltpu.sync_copy(accc, gcnt)

        def fin(i, c):
            for j in range(D // LANES):
                s = gsum[i, pl.ds(j * LANES, LANES)]
                n = gcnt[i, pl.ds(j * LANES, LANES)]
                gsum[i, pl.ds(j * LANES, LANES)] = s / jnp.maximum(n, 1.0)
            return c
        lax.fori_loop(0, G, fin, 0)
        pltpu.sync_copy(gsum, g_hbm)


def kernel(x, edge_index, batch, W1, b1, W2, b2, eps):
    ei3 = edge_index.reshape(2, NCHUNK, CH)

    agg1 = _segsum(x, ei3).reshape(2, N, D)
    cur1 = _mlp(x, agg1, W1[0], b1[0], W2[0], b2[0],
                (1.0 + eps[0]).reshape(1, 1), out_relu=True)
    agg2 = _segsum(cur1, ei3).reshape(2, N, D)
    cur2 = _mlp(cur1, agg2, W1[1], b1[1], W2[1], b2[1],
                (1.0 + eps[1]).reshape(1, 1), out_relu=True)
    agg3 = _segsum(cur2, ei3).reshape(2, N, D)
    z = _mlp3(cur2, agg3, W1[2], b1[2], W2[2], b2[2],
              (1.0 + eps[2]).reshape(1, 1), cur1, cur2)
    g = _pool(z, batch)
    return (z, g)


# speedup vs baseline: 13.1077x; 1.0610x over previous
import functools

import jax
import jax.numpy as jnp
from jax import lax
from jax.experimental import pallas as pl
from jax.experimental.pallas import tpu as pltpu
from jax.experimental.pallas import tpu_sc as plsc

N = 10000
E = 320000
D = 128
G = 64

NC = 2
NS = 16
LANES = 16

NW = NC * NS
CH = 128
NCHUNK = E // CH
CPT = NCHUNK // NW
XCH = NCHUNK - CPT * NW

ROWS_A = 624
ROWS_B = N - 15 * ROWS_A
ZR = 64


def _zero_fill(ref, nrows):
    def body(i, c):
        for j in range(D // LANES):
            ref[i, pl.ds(j * LANES, LANES)] = jnp.zeros((LANES,), jnp.float32)
        return c
    lax.fori_loop(0, nrows, body, 0)


@functools.partial(
    pl.kernel,
    out_type=jax.ShapeDtypeStruct((2 * N, D), jnp.float32),
    mesh=plsc.VectorSubcoreMesh(core_axis_name="c", subcore_axis_name="s"),
    scratch_types=[
        pltpu.VMEM((CH, D), jnp.float32),
        pltpu.VMEM((CH, D), jnp.float32),
        pltpu.VMEM((CH, D), jnp.float32),
        pltpu.VMEM((2, 1, CH), jnp.int32),
        pltpu.VMEM((2, 1, CH), jnp.int32),
        pltpu.VMEM((2, 1, CH), jnp.int32),
        pltpu.VMEM((2, 1, CH), jnp.int32),
        pltpu.VMEM_SHARED((N, D), jnp.float32),
        pltpu.SemaphoreType.DMA,
        pltpu.SemaphoreType.DMA,
        pltpu.SemaphoreType.DMA,
        pltpu.SemaphoreType.DMA,
        pltpu.SemaphoreType.DMA,
        pltpu.SemaphoreType.DMA,
        pltpu.SemaphoreType.DMA,
        pltpu.SemaphoreType.DMA,
        pltpu.SemaphoreType.DMA,
        pltpu.SemaphoreType.DMA,
    ],
)
def _segsum(x_hbm, ei_hbm, out_hbm,
            rows0, rows1, rows2, ib0, ib1, ib2, ib3, acc,
            isem0, isem1, isem2, isem3, gsem0, gsem1, gsem2,
            ssem0, ssem1, ssem2):
    cid = lax.axis_index("c")
    sid = lax.axis_index("s")
    wid = sid * NC + cid

    rows = (rows0, rows1, rows2)
    ibs = (ib0, ib1, ib2, ib3)
    isems = (isem0, isem1, isem2, isem3)
    gsems = (gsem0, gsem1, gsem2)
    ssems = (ssem0, ssem1, ssem2)

    def idx_desc(c, p4):
        return pltpu.make_async_copy(
            ei_hbm.at[:, pl.ds(c, 1), :], ibs[p4], isems[p4])

    def gather_desc(p4, p3):
        return pltpu.make_async_copy(
            x_hbm.at[ibs[p4].at[0, 0]], rows[p3], gsems[p3])

    def scatter_desc(p4, p3):
        return pltpu.make_async_copy(
            rows[p3], acc.at[ibs[p4].at[1, 0]], ssems[p3])

    _zero_fill(rows0, CH)
    rbase = sid * ROWS_A

    def zcopy(k, c):
        pltpu.sync_copy(rows0, acc.at[pl.ds(rbase + k * CH, CH)])
        return c
    lax.fori_loop(0, 4, zcopy, 0)

    @pl.when(sid == NS - 1)
    def _():
        zcopy(4, 0)

    @pl.when(sid < NS - 1)
    def _():
        pltpu.sync_copy(rows0.at[pl.ds(0, ROWS_A - 4 * CH)],
                        acc.at[pl.ds(rbase + 4 * CH, ROWS_A - 4 * CH)])

    plsc.subcore_barrier()

    cb = wid * CPT

    def body(c, j, drain=True, scat=True, fire_next=True):
        p3, p4 = j % 3, j % 4
        if drain:
            scatter_desc((j - 3) % 4, p3).wait()
        if fire_next:
            idx_desc(c + 1, (j + 1) % 4).start()
        idx_desc(c, p4).wait()
        gather_desc(p4, p3).start()
        if scat:
            gather_desc((j - 1) % 4, (j - 1) % 3).wait()
            scatter_desc((j - 1) % 4, (j - 1) % 3).start(add=True)

    idx_desc(cb, 0).start()
    body(cb + 0, 0, drain=False, scat=False)
    body(cb + 1, 1, drain=False)
    body(cb + 2, 2, drain=False)
    body(cb + 3, 3)
    body(cb + 4, 4)
    body(cb + 5, 5)

    def twelve(i, carry):
        c0 = cb + 6 + 12 * i
        for t in range(12):
            body(c0 + t, 6 + t)
        return carry
    lax.fori_loop(0, (CPT - 6) // 12, twelve, 0)

    jl = CPT - 1
    idx_desc(cb, (jl + 1) % 4).wait()
    gather_desc(jl % 4, jl % 3).wait()
    scatter_desc(jl % 4, jl % 3).start(add=True)
    scatter_desc((jl - 2) % 4, (jl - 2) % 3).wait()
    scatter_desc((jl - 1) % 4, (jl - 1) % 3).wait()
    scatter_desc(jl % 4, jl % 3).wait()

    @pl.when(wid < XCH)
    def _():
        cx = NCHUNK - XCH + wid
        idx_desc(cx, 0).start()
        idx_desc(cx, 0).wait()
        gather_desc(0, 0).start()
        gather_desc(0, 0).wait()
        scatter_desc(0, 0).start(add=True)
        scatter_desc(0, 0).wait()

    plsc.subcore_barrier()

    @pl.when(sid < NS - 1)
    def _():
        r0 = sid * ROWS_A
        pltpu.sync_copy(acc.at[pl.ds(r0, ROWS_A)],
                        out_hbm.at[pl.ds(cid * N + r0, ROWS_A)])

    @pl.when(sid == NS - 1)
    def _():
        r0 = (NS - 1) * ROWS_A
        pltpu.sync_copy(acc.at[pl.ds(r0, ROWS_B)],
                        out_hbm.at[pl.ds(cid * N + r0, ROWS_B)])


BR = 1000


def _mlp_body(scale_ref, cur_ref, agg_ref, w1_ref, b1_ref, w2_ref, b2_ref,
              out_ref, *, out_relu):
    h = scale_ref[0, 0] * cur_ref[...] + agg_ref[0] + agg_ref[1]
    t = jnp.dot(h, w1_ref[...], preferred_element_type=jnp.float32) + b1_ref[...]
    t = jnp.maximum(t, 0.0)
    o = jnp.dot(t, w2_ref[...], preferred_element_type=jnp.float32) + b2_ref[...]
    if out_relu:
        o = jnp.maximum(o, 0.0)
    out_ref[...] = o


def _mlp3_body(scale_ref, cur_ref, agg_ref, w1_ref, b1_ref, w2_ref, b2_ref,
               c1_ref, c2_ref, batch_ref, z_ref, g_ref, gsum, gcnt):
    i = pl.program_id(0)

    @pl.when(i == 0)
    def _():
        gsum[...] = jnp.zeros((G, D), jnp.float32)
        gcnt[...] = jnp.zeros((G, D), jnp.float32)

    h = scale_ref[0, 0] * cur_ref[...] + agg_ref[0] + agg_ref[1]
    t = jnp.dot(h, w1_ref[...], preferred_element_type=jnp.float32) + b1_ref[...]
    t = jnp.maximum(t, 0.0)
    o = jnp.dot(t, w2_ref[...], preferred_element_type=jnp.float32) + b2_ref[...]
    z = (c1_ref[...] + c2_ref[...] + o) * (1.0 / 3.0)
    z_ref[...] = z

    onehot = (batch_ref[...] ==
              lax.broadcasted_iota(jnp.int32, (1, G), 1)).astype(jnp.float32)
    dn = (((0,), (0,)), ((), ()))
    gsum[...] += lax.dot_general(onehot, z, dn,
                                 preferred_element_type=jnp.float32)
    gcnt[...] += lax.dot_general(onehot, jnp.ones((BR, D), jnp.float32), dn,
                                 preferred_element_type=jnp.float32)
    g_ref[...] = gsum[...] / jnp.maximum(gcnt[...], 1.0)


_scale_spec = pl.BlockSpec((1, 1), lambda i: (0, 0), memory_space=pltpu.SMEM)
_row_spec = pl.BlockSpec((BR, D), lambda i: (i, 0))
_agg_spec = pl.BlockSpec((2, BR, D), lambda i: (0, i, 0))
_w_spec = pl.BlockSpec((D, D), lambda i: (0, 0))
_b_spec = pl.BlockSpec((1, D), lambda i: (0, 0))


def _mlp(cur, agg2, w1, b1, w2, b2, scale, out_relu):
    body = functools.partial(_mlp_body, out_relu=out_relu)
    return pl.pallas_call(
        body,
        grid=(N // BR,),
        in_specs=[_scale_spec, _row_spec, _agg_spec,
                  _w_spec, _b_spec, _w_spec, _b_spec],
        out_specs=_row_spec,
        out_shape=jax.ShapeDtypeStruct((N, D), jnp.float32),
        compiler_params=pltpu.CompilerParams(
            dimension_semantics=("arbitrary",)),
    )(scale, cur, agg2, w1, b1.reshape(1, D), w2, b2.reshape(1, D))


def _mlp3(cur, agg2, w1, b1, w2, b2, scale, c1, c2, batch):
    return pl.pallas_call(
        _mlp3_body,
        grid=(N // BR,),
        in_specs=[_scale_spec, _row_spec, _agg_spec,
                  _w_spec, _b_spec, _w_spec, _b_spec,
                  _row_spec, _row_spec,
                  pl.BlockSpec((BR, 1), lambda i: (i, 0))],
        out_specs=[_row_spec, pl.BlockSpec((G, D), lambda i: (0, 0))],
        out_shape=[jax.ShapeDtypeStruct((N, D), jnp.float32),
                   jax.ShapeDtypeStruct((G, D), jnp.float32)],
        scratch_shapes=[pltpu.VMEM((G, D), jnp.float32),
                        pltpu.VMEM((G, D), jnp.float32)],
        compiler_params=pltpu.CompilerParams(
            dimension_semantics=("arbitrary",)),
    )(scale, cur, agg2, w1, b1.reshape(1, D), w2, b2.reshape(1, D), c1, c2,
      batch.reshape(N, 1))


def kernel(x, edge_index, batch, W1, b1, W2, b2, eps):
    ei3 = edge_index.reshape(2, NCHUNK, CH)

    agg1 = _segsum(x, ei3).reshape(2, N, D)
    cur1 = _mlp(x, agg1, W1[0], b1[0], W2[0], b2[0],
                (1.0 + eps[0]).reshape(1, 1), out_relu=True)
    agg2 = _segsum(cur1, ei3).reshape(2, N, D)
    cur2 = _mlp(cur1, agg2, W1[1], b1[1], W2[1], b2[1],
                (1.0 + eps[1]).reshape(1, 1), out_relu=True)
    agg3 = _segsum(cur2, ei3).reshape(2, N, D)
    z, g = _mlp3(cur2, agg3, W1[2], b1[2], W2[2], b2[2],
                 (1.0 + eps[2]).reshape(1, 1), cur1, cur2, batch)
    return (z, g)


# speedup vs baseline: 13.7590x; 1.0497x over previous
import functools

import jax
import jax.numpy as jnp
from jax import lax
from jax.experimental import pallas as pl
from jax.experimental.pallas import tpu as pltpu
from jax.experimental.pallas import tpu_sc as plsc

N = 10000
E = 320000
D = 128
G = 64

NC = 2
NS = 16
LANES = 16

NW = NC * NS
CH = 128
NCHUNK = E // CH
CPT = NCHUNK // NW
XCH = NCHUNK - CPT * NW

ROWS_A = 624
ROWS_B = N - 15 * ROWS_A
ZR = 64


def _zero_fill(ref, nrows):
    def body(i, c):
        for j in range(D // LANES):
            ref[i, pl.ds(j * LANES, LANES)] = jnp.zeros((LANES,), jnp.float32)
        return c
    lax.fori_loop(0, nrows, body, 0)


@functools.partial(
    pl.kernel,
    out_type=jax.ShapeDtypeStruct((2 * N, D), jnp.float32),
    mesh=plsc.VectorSubcoreMesh(core_axis_name="c", subcore_axis_name="s"),
    scratch_types=[
        pltpu.VMEM((CH, D), jnp.float32),
        pltpu.VMEM((CH, D), jnp.float32),
        pltpu.VMEM((CH, D), jnp.float32),
        pltpu.VMEM((2, 1, CH), jnp.int32),
        pltpu.VMEM((2, 1, CH), jnp.int32),
        pltpu.VMEM((2, 1, CH), jnp.int32),
        pltpu.VMEM((2, 1, CH), jnp.int32),
        pltpu.VMEM_SHARED((N, D), jnp.float32),
        pltpu.SemaphoreType.DMA,
        pltpu.SemaphoreType.DMA,
        pltpu.SemaphoreType.DMA,
        pltpu.SemaphoreType.DMA,
        pltpu.SemaphoreType.DMA,
        pltpu.SemaphoreType.DMA,
        pltpu.SemaphoreType.DMA,
        pltpu.SemaphoreType.DMA,
        pltpu.SemaphoreType.DMA,
        pltpu.SemaphoreType.DMA,
    ],
)
def _segsum(x_hbm, ei_hbm, out_hbm,
            rows0, rows1, rows2, ib0, ib1, ib2, ib3, acc,
            isem0, isem1, isem2, isem3, gsem0, gsem1, gsem2,
            ssem0, ssem1, ssem2):
    cid = lax.axis_index("c")
    sid = lax.axis_index("s")
    wid = sid * NC + cid

    rows = (rows0, rows1, rows2)
    ibs = (ib0, ib1, ib2, ib3)
    isems = (isem0, isem1, isem2, isem3)
    gsems = (gsem0, gsem1, gsem2)
    ssems = (ssem0, ssem1, ssem2)

    def idx_desc(c, p4):
        return pltpu.make_async_copy(
            ei_hbm.at[:, pl.ds(c, 1), :], ibs[p4], isems[p4])

    def gather_desc(p4, p3):
        return pltpu.make_async_copy(
            x_hbm.at[ibs[p4].at[0, 0]], rows[p3], gsems[p3])

    def scatter_desc(p4, p3):
        return pltpu.make_async_copy(
            rows[p3], acc.at[ibs[p4].at[1, 0]], ssems[p3])

    cb = wid * CPT

    def body(c, j, drain=True, scat=True, fire_next=True):
        p3, p4 = j % 3, j % 4
        if drain:
            scatter_desc((j - 3) % 4, p3).wait()
        if fire_next:
            idx_desc(c + 1, (j + 1) % 4).start()
        idx_desc(c, p4).wait()
        gather_desc(p4, p3).start()
        if scat:
            gather_desc((j - 1) % 4, (j - 1) % 3).wait()
            scatter_desc((j - 1) % 4, (j - 1) % 3).start(add=True)

    idx_desc(cb, 0).start()
    idx_desc(cb + 1, 1).start()
    idx_desc(cb + 2, 2).start()
    idx_desc(cb, 0).wait()
    gather_desc(0, 0).start()
    idx_desc(cb + 1, 1).wait()
    gather_desc(1, 1).start()

    _zero_fill(rows2, CH)
    rbase = sid * ROWS_A

    def zcopy(k, c):
        pltpu.sync_copy(rows2, acc.at[pl.ds(rbase + k * CH, CH)])
        return c
    lax.fori_loop(0, 4, zcopy, 0)

    @pl.when(sid == NS - 1)
    def _():
        zcopy(4, 0)

    @pl.when(sid < NS - 1)
    def _():
        pltpu.sync_copy(rows2.at[pl.ds(0, ROWS_A - 4 * CH)],
                        acc.at[pl.ds(rbase + 4 * CH, ROWS_A - 4 * CH)])

    plsc.subcore_barrier()

    gather_desc(0, 0).wait()
    scatter_desc(0, 0).start(add=True)
    body(cb + 2, 2, drain=False, scat=True)
    body(cb + 3, 3)
    body(cb + 4, 4)
    body(cb + 5, 5)

    def twelve(i, carry):
        c0 = cb + 6 + 12 * i
        for t in range(12):
            body(c0 + t, 6 + t)
        return carry
    lax.fori_loop(0, (CPT - 6) // 12, twelve, 0)

    jl = CPT - 1
    idx_desc(cb, (jl + 1) % 4).wait()
    gather_desc(jl % 4, jl % 3).wait()
    scatter_desc(jl % 4, jl % 3).start(add=True)
    scatter_desc((jl - 2) % 4, (jl - 2) % 3).wait()
    scatter_desc((jl - 1) % 4, (jl - 1) % 3).wait()
    scatter_desc(jl % 4, jl % 3).wait()

    @pl.when(wid < XCH)
    def _():
        cx = NCHUNK - XCH + wid
        idx_desc(cx, 0).start()
        idx_desc(cx, 0).wait()
        gather_desc(0, 0).start()
        gather_desc(0, 0).wait()
        scatter_desc(0, 0).start(add=True)
        scatter_desc(0, 0).wait()

    plsc.subcore_barrier()

    @pl.when(sid < NS - 1)
    def _():
        r0 = sid * ROWS_A
        pltpu.sync_copy(acc.at[pl.ds(r0, ROWS_A)],
                        out_hbm.at[pl.ds(cid * N + r0, ROWS_A)])

    @pl.when(sid == NS - 1)
    def _():
        r0 = (NS - 1) * ROWS_A
        pltpu.sync_copy(acc.at[pl.ds(r0, ROWS_B)],
                        out_hbm.at[pl.ds(cid * N + r0, ROWS_B)])


BR = 2000


def _mlp_body(scale_ref, cur_ref, agg_ref, w1_ref, b1_ref, w2_ref, b2_ref,
              out_ref, *, out_relu):
    h = scale_ref[0, 0] * cur_ref[...] + agg_ref[0] + agg_ref[1]
    t = jnp.dot(h, w1_ref[...], preferred_element_type=jnp.float32) + b1_ref[...]
    t = jnp.maximum(t, 0.0)
    o = jnp.dot(t, w2_ref[...], preferred_element_type=jnp.float32) + b2_ref[...]
    if out_relu:
        o = jnp.maximum(o, 0.0)
    out_ref[...] = o


def _mlp3_body(scale_ref, cur_ref, agg_ref, w1_ref, b1_ref, w2_ref, b2_ref,
               c1_ref, batch_ref, z_ref, g_ref, gsum, gcnt):
    i = pl.program_id(0)

    @pl.when(i == 0)
    def _():
        gsum[...] = jnp.zeros((G, D), jnp.float32)
        gcnt[...] = jnp.zeros((G, D), jnp.float32)

    h = scale_ref[0, 0] * cur_ref[...] + agg_ref[0] + agg_ref[1]
    t = jnp.dot(h, w1_ref[...], preferred_element_type=jnp.float32) + b1_ref[...]
    t = jnp.maximum(t, 0.0)
    o = jnp.dot(t, w2_ref[...], preferred_element_type=jnp.float32) + b2_ref[...]
    z = (c1_ref[...] + cur_ref[...] + o) * (1.0 / 3.0)
    z_ref[...] = z

    onehot = (batch_ref[...] ==
              lax.broadcasted_iota(jnp.int32, (1, G), 1)).astype(jnp.float32)
    dn = (((0,), (0,)), ((), ()))
    gsum[...] += lax.dot_general(onehot, z, dn,
                                 preferred_element_type=jnp.float32)
    gcnt[...] += lax.dot_general(onehot, jnp.ones((BR, D), jnp.float32), dn,
                                 preferred_element_type=jnp.float32)
    g_ref[...] = gsum[...] / jnp.maximum(gcnt[...], 1.0)


_scale_spec = pl.BlockSpec((1, 1), lambda i: (0, 0), memory_space=pltpu.SMEM)
_row_spec = pl.BlockSpec((BR, D), lambda i: (i, 0))
_agg_spec = pl.BlockSpec((2, BR, D), lambda i: (0, i, 0))
_w_spec = pl.BlockSpec((D, D), lambda i: (0, 0))
_b_spec = pl.BlockSpec((1, D), lambda i: (0, 0))


def _mlp(cur, agg2, w1, b1, w2, b2, scale, out_relu):
    body = functools.partial(_mlp_body, out_relu=out_relu)
    return pl.pallas_call(
        body,
        grid=(N // BR,),
        in_specs=[_scale_spec, _row_spec, _agg_spec,
                  _w_spec, _b_spec, _w_spec, _b_spec],
        out_specs=_row_spec,
        out_shape=jax.ShapeDtypeStruct((N, D), jnp.float32),
        compiler_params=pltpu.CompilerParams(
            dimension_semantics=("arbitrary",)),
    )(scale, cur, agg2, w1, b1.reshape(1, D), w2, b2.reshape(1, D))


def _mlp3(cur, agg2, w1, b1, w2, b2, scale, c1, batch):
    return pl.pallas_call(
        _mlp3_body,
        grid=(N // BR,),
        in_specs=[_scale_spec, _row_spec, _agg_spec,
                  _w_spec, _b_spec, _w_spec, _b_spec,
                  _row_spec,
                  pl.BlockSpec((BR, 1), lambda i: (i, 0))],
        out_specs=[_row_spec, pl.BlockSpec((G, D), lambda i: (0, 0))],
        out_shape=[jax.ShapeDtypeStruct((N, D), jnp.float32),
                   jax.ShapeDtypeStruct((G, D), jnp.float32)],
        scratch_shapes=[pltpu.VMEM((G, D), jnp.float32),
                        pltpu.VMEM((G, D), jnp.float32)],
        compiler_params=pltpu.CompilerParams(
            dimension_semantics=("arbitrary",)),
    )(scale, cur, agg2, w1, b1.reshape(1, D), w2, b2.reshape(1, D), c1,
      batch.reshape(N, 1))


def kernel(x, edge_index, batch, W1, b1, W2, b2, eps):
    ei3 = edge_index.reshape(2, NCHUNK, CH)

    agg1 = _segsum(x, ei3).reshape(2, N, D)
    cur1 = _mlp(x, agg1, W1[0], b1[0], W2[0], b2[0],
                (1.0 + eps[0]).reshape(1, 1), out_relu=True)
    agg2 = _segsum(cur1, ei3).reshape(2, N, D)
    cur2 = _mlp(cur1, agg2, W1[1], b1[1], W2[1], b2[1],
                (1.0 + eps[1]).reshape(1, 1), out_relu=True)
    agg3 = _segsum(cur2, ei3).reshape(2, N, D)
    z, g = _mlp3(cur2, agg3, W1[2], b1[2], W2[2], b2[2],
                 (1.0 + eps[2]).reshape(1, 1), cur1, batch)
    return (z, g)
